# Initial kernel scaffold; baseline (speedup 1.0000x reference)
import jax
import jax.numpy as jnp
from jax.experimental import pallas as pl


def kernel(idx_u, idx_i, edge_index, edge_vals, W_u, W_i):
    raise NotImplementedError("write your pallas kernel here")


# speedup vs baseline: 3.0237x; 3.0237x over previous
import functools

import jax
import jax.numpy as jnp
from jax import lax
from jax.experimental import pallas as pl
from jax.experimental.pallas import tpu as pltpu
from jax.experimental.pallas import tpu_sc as plsc

N_USER = 50000
N_ITEM = 50000
N_TOTAL = N_USER + N_ITEM
D = 32
E = 1600000
N_LAYER = 3
B = 16384

NC = 2
NS = 16
LANES = 16

HALF = N_TOTAL // NC
N_DUMMY = 64
ACC_ROWS = 50176
C = 128
TOTAL_CHUNKS = E // C
ZROWS = ACC_ROWS // NS
OROWS = 3128

BT = B // (NC * NS)
BC = 128

_mesh = plsc.VectorSubcoreMesh(core_axis_name="c", subcore_axis_name="s")
_params = pltpu.CompilerParams(needs_layout_passes=False,
                               use_tc_tiling_on_sc=False)


def _iota16():
    return lax.broadcasted_iota(jnp.int32, (LANES,), 0)


@functools.partial(
    pl.kernel,
    out_type=jax.ShapeDtypeStruct((N_TOTAL, D), jnp.float32),
    mesh=_mesh,
    compiler_params=_params,
    scratch_types=[
        pltpu.VMEM_SHARED((ACC_ROWS, D), jnp.float32),
        pltpu.VMEM((C,), jnp.int32),
        pltpu.VMEM((C,), jnp.int32),
        pltpu.VMEM((C // LANES, LANES), jnp.int32),
        pltpu.VMEM((C,), jnp.float32),
        pltpu.VMEM((C + LANES,), jnp.float32),
        pltpu.VMEM((C, D), jnp.float32),
    ],
)
def _layer_kernel(emb_h, src_h, dst_h, vals_h, z_h, out_h,
                  acc, sidx, didx, didx2, vstage, vbuf, rows):
    cid = lax.axis_index("c")
    sid = lax.axis_index("s")
    core_off = cid * HALF

    pltpu.sync_copy(z_h.at[pl.ds(sid * ZROWS, ZROWS)],
                    acc.at[pl.ds(sid * ZROWS, ZROWS)])
    plsc.subcore_barrier()

    nchunks = (TOTAL_CHUNKS - sid + NS - 1) // NS

    @pl.loop(0, nchunks)
    def _chunk(i):
        off = (sid + i * NS) * C
        pltpu.sync_copy(src_h.at[pl.ds(off, C)], sidx)
        pltpu.sync_copy(dst_h.at[pl.ds(off, C)], didx)
        pltpu.sync_copy(vals_h.at[pl.ds(off, C)], vstage)
        for g in range(C // LANES):
            vbuf[pl.ds(LANES + g * LANES, LANES)] = \
                vstage[pl.ds(g * LANES, LANES)]
        pltpu.sync_copy(emb_h.at[sidx], rows)
        for g in range(C // LANES):
            dv = didx[pl.ds(g * LANES, LANES)]
            dl = dv - core_off
            ok = (dl >= 0) & (dl < HALF)
            dummy = HALF + (dv & (N_DUMMY - 1))
            didx2[g, pl.ds(0, LANES)] = jnp.where(ok, dl, dummy)
        for e in range(C):
            v = plsc.load_gather(
                vbuf, [jnp.full((LANES,), e + LANES, jnp.int32)])
            rows[e, pl.ds(0, LANES)] = rows[e, pl.ds(0, LANES)] * v
            rows[e, pl.ds(LANES, LANES)] = rows[e, pl.ds(LANES, LANES)] * v
        for g in range(C // LANES):
            pltpu.sync_copy(rows.at[pl.ds(g * LANES, LANES)],
                            acc.at[didx2.at[g]], add=True)

    plsc.subcore_barrier()
    tail = HALF - (NS - 1) * OROWS

    @pl.when(sid < NS - 1)
    def _full():
        pltpu.sync_copy(acc.at[pl.ds(sid * OROWS, OROWS)],
                        out_h.at[pl.ds(core_off + sid * OROWS, OROWS)])

    @pl.when(sid == NS - 1)
    def _tail():
        pltpu.sync_copy(acc.at[pl.ds((NS - 1) * OROWS, tail)],
                        out_h.at[pl.ds(core_off + (NS - 1) * OROWS, tail)])


@functools.partial(
    pl.kernel,
    out_type=jax.ShapeDtypeStruct((B,), jnp.float32),
    mesh=_mesh,
    compiler_params=_params,
    scratch_types=[
        pltpu.VMEM((BC,), jnp.int32),
        pltpu.VMEM((BC,), jnp.int32),
        pltpu.VMEM((4, BC, D), jnp.float32),
        pltpu.VMEM((4, BC, D), jnp.float32),
        pltpu.VMEM((BC,), jnp.float32),
        pltpu.SemaphoreType.DMA,
    ],
)
def _dot_kernel(x0, x1, x2, x3, iu_h, ii_h, out_h,
                uidx, iidx, ubuf, ibuf, outv, sem):
    cid = lax.axis_index("c")
    sid = lax.axis_index("s")
    wid = cid * NS + sid

    @pl.loop(0, BT // BC)
    def _sub(s):
        base = wid * BT + s * BC
        pltpu.sync_copy(iu_h.at[pl.ds(base, BC)], uidx)
        pltpu.sync_copy(ii_h.at[pl.ds(base, BC)], iidx)
        for g in range(BC // LANES):
            iv = iidx[pl.ds(g * LANES, LANES)]
            iidx[pl.ds(g * LANES, LANES)] = iv + N_USER
        for li, xt in enumerate((x0, x1, x2, x3)):
            pltpu.sync_copy(xt.at[uidx], ubuf.at[li])
            pltpu.sync_copy(xt.at[iidx], ibuf.at[li])

        @pl.loop(0, BC // LANES)
        def _grp(g):
            rowid = jnp.full((LANES,), g * LANES, jnp.int32) + _iota16()
            acc = jnp.zeros((LANES,), jnp.float32)
            for d in range(D):
                col = jnp.full((LANES,), d, jnp.int32)
                u = plsc.load_gather(ubuf.at[0], [rowid, col])
                iv = plsc.load_gather(ibuf.at[0], [rowid, col])
                for li in range(1, 4):
                    u = u + plsc.load_gather(ubuf.at[li], [rowid, col])
                    iv = iv + plsc.load_gather(ibuf.at[li], [rowid, col])
                acc = acc + u * iv
            outv[pl.ds(g * LANES, LANES)] = acc * (1.0 / 16.0)

        pltpu.sync_copy(outv, out_h.at[pl.ds(base, BC)])


def kernel(idx_u, idx_i, edge_index, edge_vals, W_u, W_i):
    idx_u = idx_u.astype(jnp.int32)
    idx_i = idx_i.astype(jnp.int32)
    src = edge_index[0].astype(jnp.int32)
    dst = edge_index[1].astype(jnp.int32)
    emb0 = jnp.concatenate([W_u, W_i], axis=0)
    zeros = jnp.zeros((ACC_ROWS, D), jnp.float32)
    emb1 = _layer_kernel(emb0, src, dst, edge_vals, zeros)
    emb2 = _layer_kernel(emb1, src, dst, edge_vals, zeros)
    emb3 = _layer_kernel(emb2, src, dst, edge_vals, zeros)
    return _dot_kernel(emb0, emb1, emb2, emb3, idx_u, idx_i)


# speedup vs baseline: 9.1935x; 3.0404x over previous
import functools

import jax
import jax.numpy as jnp
from jax import lax
from jax.experimental import pallas as pl
from jax.experimental.pallas import tpu as pltpu
from jax.experimental.pallas import tpu_sc as plsc

N_USER = 50000
N_TOTAL = 100000
D = 32
DH = 16
E = 1600000
B = 16384

NC = 2
NS = 16
LANES = 16

C = 128
TOTAL_CHUNKS = E // C
ZROWS = N_TOTAL // NS

BT = B // (NC * NS)
BC = 128

_mesh = plsc.VectorSubcoreMesh(core_axis_name="c", subcore_axis_name="s")
_params = pltpu.CompilerParams(needs_layout_passes=False,
                               use_tc_tiling_on_sc=False)


def _iota16():
    return lax.broadcasted_iota(jnp.int32, (LANES,), 0)


@functools.partial(
    pl.kernel,
    out_type=(jax.ShapeDtypeStruct((N_TOTAL, DH), jnp.float32),
              jax.ShapeDtypeStruct((N_TOTAL, DH), jnp.float32)),
    mesh=_mesh,
    compiler_params=_params,
    scratch_types=[
        pltpu.VMEM_SHARED((N_TOTAL, DH), jnp.float32),
        pltpu.VMEM((C,), jnp.int32), pltpu.VMEM((C,), jnp.int32),
        pltpu.VMEM((C,), jnp.int32), pltpu.VMEM((C,), jnp.int32),
        pltpu.VMEM((C,), jnp.float32), pltpu.VMEM((C,), jnp.float32),
        pltpu.VMEM((C + LANES,), jnp.float32),
        pltpu.VMEM((C + LANES,), jnp.float32),
        pltpu.VMEM((C, DH), jnp.float32), pltpu.VMEM((C, DH), jnp.float32),
        pltpu.VMEM((C, DH), jnp.float32), pltpu.VMEM((C, DH), jnp.float32),
        pltpu.SemaphoreType.DMA, pltpu.SemaphoreType.DMA,
        pltpu.SemaphoreType.DMA, pltpu.SemaphoreType.DMA,
        pltpu.SemaphoreType.DMA, pltpu.SemaphoreType.DMA,
    ],
)
def _layer_kernel(embA, embB, src_h, dst_h, vals_h, z_h, outA, outB,
                  acc, sidx0, sidx1, dbuf0, dbuf1, vstage0, vstage1,
                  vbuf0, vbuf1, rows0, rows1, srows0, srows1,
                  seme0, seme1, semg0, semg1, sems0, sems1):
    cid = lax.axis_index("c")
    sid = lax.axis_index("s")

    pltpu.sync_copy(z_h.at[pl.ds(sid * ZROWS, ZROWS)],
                    acc.at[pl.ds(sid * ZROWS, ZROWS)])
    plsc.subcore_barrier()

    nchunks = (TOTAL_CHUNKS - sid + NS - 1) // NS
    sets = ((sidx0, dbuf0, vstage0, vbuf0, rows0, srows0, seme0, semg0, sems0),
            (sidx1, dbuf1, vstage1, vbuf1, rows1, srows1, seme1, semg1, sems1))

    def start_edges(i, p):
        sidx, dbuf, vstage, _, _, _, seme, _, _ = sets[p]
        off = (sid + i * NS) * C
        pltpu.async_copy(src_h.at[pl.ds(off, C)], sidx, seme)
        pltpu.async_copy(dst_h.at[pl.ds(off, C)], dbuf, seme)
        pltpu.async_copy(vals_h.at[pl.ds(off, C)], vstage, seme)

    def wait_edges(i, p):
        sidx, dbuf, vstage, _, _, _, seme, _, _ = sets[p]
        off = (sid + i * NS) * C
        pltpu.make_async_copy(src_h.at[pl.ds(off, C)], sidx, seme).wait()
        pltpu.make_async_copy(dst_h.at[pl.ds(off, C)], dbuf, seme).wait()
        pltpu.make_async_copy(vals_h.at[pl.ds(off, C)], vstage, seme).wait()

    def wait_scatter(p):
        _, dbuf, _, _, _, srows, _, _, sems = sets[p]
        pltpu.make_async_copy(srows, acc.at[dbuf], sems).wait()

    def body(i, p):
        sidx, dbuf, vstage, vbuf, rows, srows, seme, semg, sems = sets[p]
        wait_edges(i, p)

        @pl.when(cid == 0)
        def _ga():
            pltpu.async_copy(embA.at[sidx], rows, semg)

        @pl.when(cid == 1)
        def _gb():
            pltpu.async_copy(embB.at[sidx], rows, semg)

        @pl.when(i >= 1)
        def _ws():
            wait_scatter(1 - p)

        @pl.when(i + 1 < nchunks)
        def _pre():
            start_edges(i + 1, 1 - p)

        @pl.when(cid == 0)
        def _gaw():
            pltpu.make_async_copy(embA.at[sidx], rows, semg).wait()

        @pl.when(cid == 1)
        def _gbw():
            pltpu.make_async_copy(embB.at[sidx], rows, semg).wait()

        for g in range(C // LANES):
            vbuf[pl.ds(LANES + g * LANES, LANES)] = \
                vstage[pl.ds(g * LANES, LANES)]
        for e in range(0, C, 4):
            vv = [plsc.load_gather(
                vbuf, [jnp.full((LANES,), e + k + LANES, jnp.int32)])
                for k in range(4)]
            rr = [rows[e + k, pl.ds(0, LANES)] for k in range(4)]
            for k in range(4):
                srows[e + k, pl.ds(0, LANES)] = rr[k] * vv[k]

        pltpu.async_copy(srows, acc.at[dbuf], sems, add=True)

    start_edges(0, 0)

    @pl.loop(0, nchunks // 2)
    def _pair(j):
        body(2 * j, 0)
        body(2 * j + 1, 1)

    @pl.when(nchunks % 2 == 1)
    def _odd():
        body(nchunks - 1, 0)
        wait_scatter(0)

    @pl.when(nchunks % 2 == 0)
    def _even():
        wait_scatter(1)

    plsc.subcore_barrier()

    @pl.when(cid == 0)
    def _wa():
        pltpu.sync_copy(acc.at[pl.ds(sid * ZROWS, ZROWS)],
                        outA.at[pl.ds(sid * ZROWS, ZROWS)])

    @pl.when(cid == 1)
    def _wb():
        pltpu.sync_copy(acc.at[pl.ds(sid * ZROWS, ZROWS)],
                        outB.at[pl.ds(sid * ZROWS, ZROWS)])


@functools.partial(
    pl.kernel,
    out_type=jax.ShapeDtypeStruct((B,), jnp.float32),
    mesh=_mesh,
    compiler_params=_params,
    scratch_types=[
        pltpu.VMEM((BC,), jnp.int32),
        pltpu.VMEM((BC,), jnp.int32),
        pltpu.VMEM((8, BC, DH), jnp.float32),
        pltpu.VMEM((8, BC, DH), jnp.float32),
        pltpu.VMEM((BC,), jnp.float32),
    ],
)
def _dot_kernel(x0A, x1A, x2A, x3A, x0B, x1B, x2B, x3B, iu_h, ii_h, out_h,
                uidx, iidx, ubuf, ibuf, outv):
    cid = lax.axis_index("c")
    sid = lax.axis_index("s")
    wid = cid * NS + sid

    @pl.loop(0, BT // BC)
    def _sub(s):
        base = wid * BT + s * BC
        pltpu.sync_copy(iu_h.at[pl.ds(base, BC)], uidx)
        pltpu.sync_copy(ii_h.at[pl.ds(base, BC)], iidx)
        for g in range(BC // LANES):
            iv = iidx[pl.ds(g * LANES, LANES)]
            iidx[pl.ds(g * LANES, LANES)] = iv + N_USER
        for li, xt in enumerate((x0A, x1A, x2A, x3A, x0B, x1B, x2B, x3B)):
            pltpu.sync_copy(xt.at[uidx], ubuf.at[li])
            pltpu.sync_copy(xt.at[iidx], ibuf.at[li])

        @pl.loop(0, BC // LANES)
        def _grp(g):
            rowid = jnp.full((LANES,), g * LANES, jnp.int32) + _iota16()
            acc = jnp.zeros((LANES,), jnp.float32)
            for half in range(2):
                for d in range(DH):
                    col = jnp.full((LANES,), d, jnp.int32)
                    u = plsc.load_gather(ubuf.at[4 * half], [rowid, col])
                    iv = plsc.load_gather(ibuf.at[4 * half], [rowid, col])
                    for li in range(1, 4):
                        u = u + plsc.load_gather(
                            ubuf.at[4 * half + li], [rowid, col])
                        iv = iv + plsc.load_gather(
                            ibuf.at[4 * half + li], [rowid, col])
                    acc = acc + u * iv
            outv[pl.ds(g * LANES, LANES)] = acc * (1.0 / 16.0)

        pltpu.sync_copy(outv, out_h.at[pl.ds(base, BC)])


def kernel(idx_u, idx_i, edge_index, edge_vals, W_u, W_i):
    idx_u = idx_u.astype(jnp.int32)
    idx_i = idx_i.astype(jnp.int32)
    src = edge_index[0].astype(jnp.int32)
    dst = edge_index[1].astype(jnp.int32)
    emb0 = jnp.concatenate([W_u, W_i], axis=0)
    e0A = emb0[:, :DH]
    e0B = emb0[:, DH:]
    zeros = jnp.zeros((N_TOTAL, DH), jnp.float32)
    e1A, e1B = _layer_kernel(e0A, e0B, src, dst, edge_vals, zeros)
    e2A, e2B = _layer_kernel(e1A, e1B, src, dst, edge_vals, zeros)
    e3A, e3B = _layer_kernel(e2A, e2B, src, dst, edge_vals, zeros)
    return _dot_kernel(e0A, e1A, e2A, e3A, e0B, e1B, e2B, e3B,
                       idx_u, idx_i)


# speedup vs baseline: 12.0122x; 1.3066x over previous
import functools

import jax
import jax.numpy as jnp
from jax import lax
from jax.experimental import pallas as pl
from jax.experimental.pallas import tpu as pltpu
from jax.experimental.pallas import tpu_sc as plsc

N_USER = 50000
N_TOTAL = 100000
D = 32
DH = 16
E = 1600000
B = 16384

NC = 2
NS = 16
LANES = 16

C = 128
TOTAL_CHUNKS = E // C
ZROWS = N_TOTAL // NS

BT = B // (NC * NS)
BC = 128

_mesh = plsc.VectorSubcoreMesh(core_axis_name="c", subcore_axis_name="s")
_params = pltpu.CompilerParams(needs_layout_passes=False,
                               use_tc_tiling_on_sc=False)


def _iota16():
    return lax.broadcasted_iota(jnp.int32, (LANES,), 0)


@functools.partial(
    pl.kernel,
    out_type=(jax.ShapeDtypeStruct((N_TOTAL, DH), jnp.float32),
              jax.ShapeDtypeStruct((N_TOTAL, DH), jnp.float32)),
    mesh=_mesh,
    compiler_params=_params,
    scratch_types=[
        pltpu.VMEM_SHARED((N_TOTAL, DH), jnp.float32),
        pltpu.VMEM((3, C), jnp.int32),
        pltpu.VMEM((3, C), jnp.int32),
        pltpu.VMEM((3, C), jnp.float32),
        pltpu.VMEM((3, C + LANES), jnp.float32),
        pltpu.VMEM((C, DH), jnp.float32), pltpu.VMEM((C, DH), jnp.float32),
        pltpu.VMEM((C, DH), jnp.float32), pltpu.VMEM((C, DH), jnp.float32),
        pltpu.SemaphoreType.DMA, pltpu.SemaphoreType.DMA,
        pltpu.SemaphoreType.DMA,
        pltpu.SemaphoreType.DMA, pltpu.SemaphoreType.DMA,
        pltpu.SemaphoreType.DMA, pltpu.SemaphoreType.DMA,
    ],
)
def _layer_kernel(embA, embB, src_h, dst_h, vals_h, z_h, outA, outB,
                  acc, sidx, dbuf, vstage, vbuf,
                  rows0, rows1, srows0, srows1,
                  seme0, seme1, seme2, semg0, semg1, sems0, sems1):
    cid = lax.axis_index("c")
    sid = lax.axis_index("s")

    pltpu.sync_copy(z_h.at[pl.ds(sid * ZROWS, ZROWS)],
                    acc.at[pl.ds(sid * ZROWS, ZROWS)])
    plsc.subcore_barrier()

    nchunks = (TOTAL_CHUNKS - sid + NS - 1) // NS
    semes = (seme0, seme1, seme2)
    rowss = (rows0, rows1)
    srowss = (srows0, srows1)
    semgs = (semg0, semg1)
    semss = (sems0, sems1)

    def start_edges(i, q):
        off = (sid + i * NS) * C
        pltpu.async_copy(src_h.at[pl.ds(off, C)], sidx.at[q], semes[q])
        pltpu.async_copy(dst_h.at[pl.ds(off, C)], dbuf.at[q], semes[q])
        pltpu.async_copy(vals_h.at[pl.ds(off, C)], vstage.at[q], semes[q])

    def wait_edges(i, q):
        off = (sid + i * NS) * C
        pltpu.make_async_copy(
            src_h.at[pl.ds(off, C)], sidx.at[q], semes[q]).wait()
        pltpu.make_async_copy(
            dst_h.at[pl.ds(off, C)], dbuf.at[q], semes[q]).wait()
        pltpu.make_async_copy(
            vals_h.at[pl.ds(off, C)], vstage.at[q], semes[q]).wait()

    def start_gather(p, q):
        @pl.when(cid == 0)
        def _a():
            pltpu.async_copy(embA.at[sidx.at[q]], rowss[p], semgs[p])

        @pl.when(cid == 1)
        def _b():
            pltpu.async_copy(embB.at[sidx.at[q]], rowss[p], semgs[p])

    def wait_gather(p, q):
        @pl.when(cid == 0)
        def _a():
            pltpu.make_async_copy(
                embA.at[sidx.at[q]], rowss[p], semgs[p]).wait()

        @pl.when(cid == 1)
        def _b():
            pltpu.make_async_copy(
                embB.at[sidx.at[q]], rowss[p], semgs[p]).wait()

    def wait_scatter(p, q):
        pltpu.make_async_copy(
            srowss[p], acc.at[dbuf.at[q]], semss[p]).wait()

    def body(i, p, q):
        q1 = (q + 1) % 3
        q2 = (q + 2) % 3
        rows, srows = rowss[p], srowss[p]

        @pl.when(i + 1 < nchunks)
        def _next_gather():
            wait_edges(i + 1, q1)
            start_gather(1 - p, q1)

        wait_gather(p, q)

        @pl.when(i >= 1)
        def _ws():
            wait_scatter(1 - p, q2)

        @pl.when(i + 2 < nchunks)
        def _pre():
            start_edges(i + 2, q2)

        for g in range(C // LANES):
            vbuf[q, pl.ds(LANES + g * LANES, LANES)] = \
                vstage[q, pl.ds(g * LANES, LANES)]
        for e in range(0, C, 4):
            vv = [plsc.load_gather(
                vbuf.at[q], [jnp.full((LANES,), e + k + LANES, jnp.int32)])
                for k in range(4)]
            rr = [rows[e + k, pl.ds(0, LANES)] for k in range(4)]
            for k in range(4):
                srows[e + k, pl.ds(0, LANES)] = rr[k] * vv[k]

        pltpu.async_copy(srows, acc.at[dbuf.at[q]], semss[p], add=True)

    start_edges(0, 0)
    start_edges(1, 1)
    wait_edges(0, 0)
    start_gather(0, 0)

    @pl.loop(0, nchunks // 6)
    def _six(j):
        for t in range(6):
            body(6 * j + t, t % 2, t % 3)

    base = (nchunks // 6) * 6
    for t in range(2):
        @pl.when(base + t < nchunks)
        def _tail(t=t):
            body(base + t, t % 2, t % 3)

    @pl.when(nchunks % 6 == 1)
    def _drain1():
        wait_scatter(0, 0)

    @pl.when(nchunks % 6 == 2)
    def _drain2():
        wait_scatter(1, 1)

    plsc.subcore_barrier()

    @pl.when(cid == 0)
    def _wa():
        pltpu.sync_copy(acc.at[pl.ds(sid * ZROWS, ZROWS)],
                        outA.at[pl.ds(sid * ZROWS, ZROWS)])

    @pl.when(cid == 1)
    def _wb():
        pltpu.sync_copy(acc.at[pl.ds(sid * ZROWS, ZROWS)],
                        outB.at[pl.ds(sid * ZROWS, ZROWS)])


@functools.partial(
    pl.kernel,
    out_type=jax.ShapeDtypeStruct((B,), jnp.float32),
    mesh=_mesh,
    compiler_params=_params,
    scratch_types=[
        pltpu.VMEM((BC,), jnp.int32),
        pltpu.VMEM((BC,), jnp.int32),
        pltpu.VMEM((8, BC, DH), jnp.float32),
        pltpu.VMEM((8, BC, DH), jnp.float32),
        pltpu.VMEM((BC,), jnp.float32),
    ],
)
def _dot_kernel(x0A, x1A, x2A, x3A, x0B, x1B, x2B, x3B, iu_h, ii_h, out_h,
                uidx, iidx, ubuf, ibuf, outv):
    cid = lax.axis_index("c")
    sid = lax.axis_index("s")
    wid = cid * NS + sid

    @pl.loop(0, BT // BC)
    def _sub(s):
        base = wid * BT + s * BC
        pltpu.sync_copy(iu_h.at[pl.ds(base, BC)], uidx)
        pltpu.sync_copy(ii_h.at[pl.ds(base, BC)], iidx)
        for g in range(BC // LANES):
            iv = iidx[pl.ds(g * LANES, LANES)]
            iidx[pl.ds(g * LANES, LANES)] = iv + N_USER
        for li, xt in enumerate((x0A, x1A, x2A, x3A, x0B, x1B, x2B, x3B)):
            pltpu.sync_copy(xt.at[uidx], ubuf.at[li])
            pltpu.sync_copy(xt.at[iidx], ibuf.at[li])

        @pl.loop(0, BC // LANES)
        def _grp(g):
            rowid = jnp.full((LANES,), g * LANES, jnp.int32) + _iota16()
            acc = jnp.zeros((LANES,), jnp.float32)
            for half in range(2):
                for d in range(DH):
                    col = jnp.full((LANES,), d, jnp.int32)
                    u = plsc.load_gather(ubuf.at[4 * half], [rowid, col])
                    iv = plsc.load_gather(ibuf.at[4 * half], [rowid, col])
                    for li in range(1, 4):
                        u = u + plsc.load_gather(
                            ubuf.at[4 * half + li], [rowid, col])
                        iv = iv + plsc.load_gather(
                            ibuf.at[4 * half + li], [rowid, col])
                    acc = acc + u * iv
            outv[pl.ds(g * LANES, LANES)] = acc * (1.0 / 16.0)

        pltpu.sync_copy(outv, out_h.at[pl.ds(base, BC)])


def kernel(idx_u, idx_i, edge_index, edge_vals, W_u, W_i):
    idx_u = idx_u.astype(jnp.int32)
    idx_i = idx_i.astype(jnp.int32)
    src = edge_index[0].astype(jnp.int32)
    dst = edge_index[1].astype(jnp.int32)
    emb0 = jnp.concatenate([W_u, W_i], axis=0)
    e0A = emb0[:, :DH]
    e0B = emb0[:, DH:]
    zeros = jnp.zeros((N_TOTAL, DH), jnp.float32)
    e1A, e1B = _layer_kernel(e0A, e0B, src, dst, edge_vals, zeros)
    e2A, e2B = _layer_kernel(e1A, e1B, src, dst, edge_vals, zeros)
    e3A, e3B = _layer_kernel(e2A, e2B, src, dst, edge_vals, zeros)
    return _dot_kernel(e0A, e1A, e2A, e3A, e0B, e1B, e2B, e3B,
                       idx_u, idx_i)
